# Initial kernel scaffold; baseline (speedup 1.0000x reference)
import jax
import jax.numpy as jnp
from jax.experimental import pallas as pl


def kernel(edge_index, W, b):
    raise NotImplementedError("write your pallas kernel here")


# speedup vs baseline: 7.3131x; 7.3131x over previous
import jax
import jax.numpy as jnp
from jax import lax
from jax.experimental import pallas as pl
from jax.experimental.pallas import tpu as pltpu
from jax.experimental.pallas import tpu_sc as plsc

N = 10000
NPAD = 10240
C = 128
E = 320000
NC = 2
NS = 16
NW = NC * NS
EPW = E // NW
K = 80
NCH = EPW // K
RPT = NPAD // NS
RCH = 64
NRCH = RPT // RCH
L = 16
IMAX = 2147483647


def _sc_body(rows_hbm, cols_hbm, wt_hbm, zeros_hbm, parts_hbm,
             row_v, col_v, ridx_v, cidx_v, grow_v, stage_v,
             minvec_v, minsall_v, mins_sh, acc_sh, gsem):
    c = lax.axis_index("c")
    s = lax.axis_index("s")
    wid = c * NS + s
    mirror = (1 - c) * NS + s

    pltpu.sync_copy(rows_hbm.at[pl.ds(wid * EPW, EPW)], row_v)
    pltpu.sync_copy(rows_hbm.at[pl.ds(mirror * EPW, EPW)], col_v)

    def minbody(i, mv):
        a = row_v[pl.ds(i * L, L)]
        b2 = col_v[pl.ds(i * L, L)]
        return jnp.minimum(mv, jnp.minimum(a, b2))

    mv = lax.fori_loop(0, EPW // L, minbody, jnp.full((L,), IMAX, jnp.int32))
    minvec_v[...] = mv
    pltpu.sync_copy(minvec_v, mins_sh.at[s])
    pltpu.sync_copy(cols_hbm.at[pl.ds(wid * EPW, EPW)], col_v)

    r0 = s * RPT
    for k in range(NRCH):
        st = r0 + k * RCH
        pltpu.sync_copy(zeros_hbm.at[pl.ds(st, RCH)], stage_v)
        pltpu.sync_copy(stage_v, acc_sh.at[pl.ds(st, RCH)])

    plsc.subcore_barrier()

    pltpu.sync_copy(mins_sh, minsall_v)
    mv2 = minsall_v[0]
    for t in range(1, NS):
        mv2 = jnp.minimum(mv2, minsall_v[t])
    m = mv2[0]
    for t in range(1, L):
        m = jnp.minimum(m, mv2[t])

    def chunk_body(cc, _):
        base = cc * K
        for j in range(K // L):
            off = j * L
            ridx_v[pl.ds(off, L)] = row_v[pl.ds(base + off, L)] - m
            cidx_v[pl.ds(off, L)] = col_v[pl.ds(base + off, L)]
        pltpu.async_copy(wt_hbm.at[cidx_v], grow_v, gsem).wait()
        pltpu.sync_copy(grow_v, acc_sh.at[ridx_v], add=True)
        return 0

    lax.fori_loop(0, NCH, chunk_body, 0)

    plsc.subcore_barrier()

    for k in range(NRCH):
        st = r0 + k * RCH
        pltpu.sync_copy(acc_sh.at[pl.ds(st, RCH)], stage_v)
        pltpu.sync_copy(stage_v, parts_hbm.at[c, pl.ds(st, RCH)])


_sc_call = pl.kernel(
    _sc_body,
    out_type=jax.ShapeDtypeStruct((NC, NPAD, C), jnp.float32),
    mesh=plsc.VectorSubcoreMesh(core_axis_name="c", subcore_axis_name="s"),
    scratch_types=[
        pltpu.VMEM((EPW,), jnp.int32),
        pltpu.VMEM((EPW,), jnp.int32),
        pltpu.VMEM((K,), jnp.int32),
        pltpu.VMEM((K,), jnp.int32),
        pltpu.VMEM((K, C), jnp.float32),
        pltpu.VMEM((RCH, C), jnp.float32),
        pltpu.VMEM((L,), jnp.int32),
        pltpu.VMEM((NS, L), jnp.int32),
        pltpu.VMEM_SHARED((NS, L), jnp.int32),
        pltpu.VMEM_SHARED((NPAD, C), jnp.float32),
        pltpu.SemaphoreType.DMA,
    ],
)


def _merge_body(p_ref, b_ref, o_ref):
    o_ref[...] = p_ref[0] + p_ref[1] + b_ref[...]


def _merge(parts, b):
    rb = 2000
    return pl.pallas_call(
        _merge_body,
        grid=(N // rb,),
        in_specs=[
            pl.BlockSpec((NC, rb, C), lambda i: (0, i, 0)),
            pl.BlockSpec((1, C), lambda i: (0, 0)),
        ],
        out_specs=pl.BlockSpec((rb, C), lambda i: (i, 0)),
        out_shape=jax.ShapeDtypeStruct((N, C), jnp.float32),
    )(parts, b.reshape(1, C))


@jax.jit
def _impl(edge_index, W, b):
    row = edge_index[0].astype(jnp.int32).reshape(E)
    col = edge_index[1].astype(jnp.int32).reshape(E)
    wt = W.T.reshape(N, C)
    zeros = jnp.zeros((NPAD, C), jnp.float32)
    parts = _sc_call(row, col, wt, zeros)
    return _merge(parts, b)


def kernel(edge_index, W, b):
    return _impl(edge_index, W, b)


# speedup vs baseline: 10.8872x; 1.4887x over previous
import jax
import jax.numpy as jnp
from jax import lax
from jax.experimental import pallas as pl
from jax.experimental.pallas import tpu as pltpu
from jax.experimental.pallas import tpu_sc as plsc

N = 10000
NPAD = 10240
C = 128
E = 320000
NC = 2
NS = 16
NW = NC * NS
EPW = E // NW
K = 80
NCH = EPW // K
RPT = NPAD // NS
RCH = 40
NRCH = RPT // RCH
L = 16
IMAX = 2147483647


def _sc_body(rows_hbm, cols_hbm, wt_hbm, zeros_hbm, parts_hbm,
             row_v, col_v, ridx0_v, cidx0_v, ridx1_v, cidx1_v,
             grow0_v, grow1_v, stage_v,
             minvec_v, minsall_v, mins_sh, acc_sh,
             g0sem, g1sem, s0sem, s1sem):
    c = lax.axis_index("c")
    s = lax.axis_index("s")
    wid = c * NS + s
    mirror = (1 - c) * NS + s

    pltpu.sync_copy(rows_hbm.at[pl.ds(wid * EPW, EPW)], row_v)
    pltpu.sync_copy(rows_hbm.at[pl.ds(mirror * EPW, EPW)], col_v)

    def minbody(i, mv):
        a = row_v[pl.ds(i * L, L)]
        b2 = col_v[pl.ds(i * L, L)]
        return jnp.minimum(mv, jnp.minimum(a, b2))

    mv = lax.fori_loop(0, EPW // L, minbody, jnp.full((L,), IMAX, jnp.int32))
    minvec_v[...] = mv
    pltpu.sync_copy(minvec_v, mins_sh.at[s])
    pltpu.sync_copy(cols_hbm.at[pl.ds(wid * EPW, EPW)], col_v)

    r0 = s * RPT
    for k in range(NRCH):
        st = r0 + k * RCH
        pltpu.sync_copy(zeros_hbm.at[pl.ds(st, RCH)], stage_v)
        pltpu.sync_copy(stage_v, acc_sh.at[pl.ds(st, RCH)])

    plsc.subcore_barrier()

    pltpu.sync_copy(mins_sh, minsall_v)
    mv2 = minsall_v[0]
    for t in range(1, NS):
        mv2 = jnp.minimum(mv2, minsall_v[t])
    m = mv2[0]
    for t in range(1, L):
        m = jnp.minimum(m, mv2[t])

    def stage(cidx, ridx, cc):
        base = cc * K
        for j in range(K // L):
            off = j * L
            ridx[pl.ds(off, L)] = row_v[pl.ds(base + off, L)] - m
            cidx[pl.ds(off, L)] = col_v[pl.ds(base + off, L)]

    def gstart(cidx, grow, sem):
        pltpu.async_copy(wt_hbm.at[cidx], grow, sem)

    def gwait(cidx, grow, sem):
        pltpu.make_async_copy(wt_hbm.at[cidx], grow, sem).wait()

    def sstart(grow, ridx, sem):
        pltpu.async_copy(grow, acc_sh.at[ridx], sem, add=True)

    def swait(grow, ridx, sem):
        pltpu.make_async_copy(grow, acc_sh.at[ridx], sem).wait()

    stage(cidx0_v, ridx0_v, 0)
    gstart(cidx0_v, grow0_v, g0sem)
    gwait(cidx0_v, grow0_v, g0sem)
    sstart(grow0_v, ridx0_v, s0sem)
    stage(cidx1_v, ridx1_v, 1)
    gstart(cidx1_v, grow1_v, g1sem)
    swait(grow0_v, ridx0_v, s0sem)
    stage(cidx0_v, ridx0_v, 2)
    gstart(cidx0_v, grow0_v, g0sem)
    gwait(cidx1_v, grow1_v, g1sem)
    sstart(grow1_v, ridx1_v, s1sem)

    def pair_body(g, _):
        c0 = 2 * g
        swait(grow1_v, ridx1_v, s1sem)
        stage(cidx1_v, ridx1_v, c0 + 1)
        gstart(cidx1_v, grow1_v, g1sem)
        gwait(cidx0_v, grow0_v, g0sem)
        sstart(grow0_v, ridx0_v, s0sem)
        swait(grow0_v, ridx0_v, s0sem)
        stage(cidx0_v, ridx0_v, c0 + 2)
        gstart(cidx0_v, grow0_v, g0sem)
        gwait(cidx1_v, grow1_v, g1sem)
        sstart(grow1_v, ridx1_v, s1sem)
        return 0

    lax.fori_loop(1, (NCH - 1) // 2, pair_body, 0)

    gwait(cidx0_v, grow0_v, g0sem)
    sstart(grow0_v, ridx0_v, s0sem)
    swait(grow1_v, ridx1_v, s1sem)
    swait(grow0_v, ridx0_v, s0sem)

    plsc.subcore_barrier()

    for k in range(NRCH):
        st = r0 + k * RCH
        pltpu.sync_copy(acc_sh.at[pl.ds(st, RCH)], stage_v)
        pltpu.sync_copy(stage_v, parts_hbm.at[c, pl.ds(st, RCH)])


_sc_call = pl.kernel(
    _sc_body,
    out_type=jax.ShapeDtypeStruct((NC, NPAD, C), jnp.float32),
    mesh=plsc.VectorSubcoreMesh(core_axis_name="c", subcore_axis_name="s"),
    scratch_types=[
        pltpu.VMEM((EPW,), jnp.int32),
        pltpu.VMEM((EPW,), jnp.int32),
        pltpu.VMEM((K,), jnp.int32),
        pltpu.VMEM((K,), jnp.int32),
        pltpu.VMEM((K,), jnp.int32),
        pltpu.VMEM((K,), jnp.int32),
        pltpu.VMEM((K, C), jnp.float32),
        pltpu.VMEM((K, C), jnp.float32),
        pltpu.VMEM((RCH, C), jnp.float32),
        pltpu.VMEM((L,), jnp.int32),
        pltpu.VMEM((NS, L), jnp.int32),
        pltpu.VMEM_SHARED((NS, L), jnp.int32),
        pltpu.VMEM_SHARED((NPAD, C), jnp.float32),
        pltpu.SemaphoreType.DMA,
        pltpu.SemaphoreType.DMA,
        pltpu.SemaphoreType.DMA,
        pltpu.SemaphoreType.DMA,
    ],
)


def _merge_body(p_ref, b_ref, o_ref):
    o_ref[...] = p_ref[0] + p_ref[1] + b_ref[...]


def _merge(parts, b):
    rb = 2000
    return pl.pallas_call(
        _merge_body,
        grid=(N // rb,),
        in_specs=[
            pl.BlockSpec((NC, rb, C), lambda i: (0, i, 0)),
            pl.BlockSpec((1, C), lambda i: (0, 0)),
        ],
        out_specs=pl.BlockSpec((rb, C), lambda i: (i, 0)),
        out_shape=jax.ShapeDtypeStruct((N, C), jnp.float32),
    )(parts, b.reshape(1, C))


@jax.jit
def _impl(edge_index, W, b):
    row = edge_index[0].astype(jnp.int32).reshape(E)
    col = edge_index[1].astype(jnp.int32).reshape(E)
    wt = W.T.reshape(N, C)
    zeros = jnp.zeros((NPAD, C), jnp.float32)
    parts = _sc_call(row, col, wt, zeros)
    return _merge(parts, b)


def kernel(edge_index, W, b):
    return _impl(edge_index, W, b)
